# Initial kernel scaffold; baseline (speedup 1.0000x reference)
import functools

import jax
import jax.numpy as jnp
from jax import lax
from jax.experimental import pallas as pl
from jax.experimental.pallas import tpu as pltpu
from jax.experimental.pallas import tpu_sc as plsc

N = 10000
E = 320000
D = 128
H = 32
OUT = 2

NC = 2
NS = 16
LANES = 16
NW = NC * NS

BATCH = 128
CH = 80
EDGES_PER_TILE = CH * BATCH
E_PAD = NW * EDGES_PER_TILE
N_PAD = 10240
RPT = N_PAD // NS

_mesh = plsc.VectorSubcoreMesh(
    core_axis_name="c", subcore_axis_name="s", num_cores=NC, num_subcores=NS
)


@functools.partial(
    pl.kernel,
    out_type=jax.ShapeDtypeStruct((NC, N_PAD), jnp.float32),
    mesh=_mesh,
    scratch_types=[
        pltpu.VMEM((CH, BATCH), jnp.int32),
        pltpu.VMEM((N_PAD,), jnp.float32),
        pltpu.VMEM_SHARED((N_PAD,), jnp.float32),
    ],
)
def _deg_kernel(dst_hbm, out_hbm, dstbuf, degloc, degsh):
    c = lax.axis_index("c")
    s = lax.axis_index("s")
    gid = c * NS + s

    pltpu.sync_copy(dst_hbm.at[gid], dstbuf)

    zeros16 = jnp.zeros((LANES,), jnp.float32)

    def _zero(i, _):
        degloc[pl.ds(i * LANES, LANES)] = zeros16
        return 0

    lax.fori_loop(0, N_PAD // LANES, _zero, 0)
    pltpu.sync_copy(degloc.at[pl.ds(s * RPT, RPT)], degsh.at[pl.ds(s * RPT, RPT)])
    plsc.subcore_barrier()

    ones16 = jnp.ones((LANES,), jnp.float32)

    def _count(i, _):
        for j in range(BATCH // LANES):
            idx = dstbuf[i, pl.ds(j * LANES, LANES)]
            plsc.addupdate_scatter(degloc, [idx], ones16)
        return 0

    lax.fori_loop(0, CH, _count, 0)

    pltpu.sync_copy(degloc, degsh, add=True)
    plsc.subcore_barrier()
    pltpu.sync_copy(degsh.at[pl.ds(s * RPT, RPT)], out_hbm.at[c, pl.ds(s * RPT, RPT)])


@functools.partial(
    pl.kernel,
    out_type=jax.ShapeDtypeStruct((NC, N_PAD, H), jnp.float32),
    mesh=_mesh,
    scratch_types=[
        pltpu.VMEM((CH, BATCH), jnp.int32),
        pltpu.VMEM((CH, BATCH), jnp.int32),
        pltpu.VMEM((BATCH, H), jnp.float32),
        pltpu.VMEM((BATCH, H), jnp.float32),
        pltpu.VMEM((RPT, H), jnp.float32),
        pltpu.VMEM_SHARED((N_PAD, H), jnp.float32),
        pltpu.SemaphoreType.DMA,
        pltpu.SemaphoreType.DMA,
    ],
)
def _agg_kernel(g_hbm, src_hbm, dst_hbm, out_hbm,
                srcbuf, dstbuf, rows0, rows1, zbuf, acc, sem0, sem1):
    c = lax.axis_index("c")
    s = lax.axis_index("s")
    gid = c * NS + s

    pltpu.sync_copy(src_hbm.at[gid], srcbuf)
    pltpu.sync_copy(dst_hbm.at[gid], dstbuf)

    zeros16 = jnp.zeros((LANES,), jnp.float32)

    def _zero(i, _):
        zbuf[i, pl.ds(0, LANES)] = zeros16
        zbuf[i, pl.ds(LANES, LANES)] = zeros16
        return 0

    lax.fori_loop(0, RPT, _zero, 0)
    pltpu.sync_copy(zbuf, acc.at[pl.ds(s * RPT, RPT)])
    plsc.subcore_barrier()

    pltpu.async_copy(g_hbm.at[srcbuf.at[0]], rows0, sem0).wait()

    def _step(k, _):
        pltpu.async_copy(g_hbm.at[srcbuf.at[2 * k + 1]], rows1, sem1)
        pltpu.sync_copy(rows0, acc.at[dstbuf.at[2 * k]], add=True)
        pltpu.make_async_copy(g_hbm.at[srcbuf.at[2 * k + 1]], rows1, sem1).wait()
        pltpu.async_copy(g_hbm.at[srcbuf.at[2 * k + 2]], rows0, sem0)
        pltpu.sync_copy(rows1, acc.at[dstbuf.at[2 * k + 1]], add=True)
        pltpu.make_async_copy(g_hbm.at[srcbuf.at[2 * k + 2]], rows0, sem0).wait()
        return 0

    lax.fori_loop(0, CH // 2 - 1, _step, 0)
    pltpu.async_copy(g_hbm.at[srcbuf.at[CH - 1]], rows1, sem1)
    pltpu.sync_copy(rows0, acc.at[dstbuf.at[CH - 2]], add=True)
    pltpu.make_async_copy(g_hbm.at[srcbuf.at[CH - 1]], rows1, sem1).wait()
    pltpu.sync_copy(rows1, acc.at[dstbuf.at[CH - 1]], add=True)

    plsc.subcore_barrier()
    pltpu.sync_copy(acc.at[pl.ds(s * RPT, RPT)], out_hbm.at[c, pl.ds(s * RPT, RPT)])


def _dis_from(degp_ref):
    d = degp_ref[:, 0:1] + degp_ref[:, 1:2] + 1.0
    return lax.rsqrt(d)


def _tc_a_body(x_ref, w1_ref, degp_ref, g1_ref):
    dis = _dis_from(degp_ref)
    h = jnp.dot(x_ref[...], w1_ref[...], preferred_element_type=jnp.float32)
    g1_ref[...] = dis * h


def _tc_b_body(degp_ref, g1_ref, s1_ref, b1_ref, q_ref):
    dis = _dis_from(degp_ref)
    ssum = s1_ref[0, :N, :] + s1_ref[1, :N, :]
    out1 = dis * (ssum + g1_ref[...]) + b1_ref[...]
    q_ref[...] = dis * jnp.maximum(out1, 0.0)


def _tc_c_body(degp_ref, q_ref, r_ref, w2_ref, b2_ref, o_ref):
    dis = _dis_from(degp_ref)
    rsum = r_ref[0, :N, :] + r_ref[1, :N, :]
    t = dis * (rsum + q_ref[...])
    o = jnp.dot(t, w2_ref[...], preferred_element_type=jnp.float32) + b2_ref[...]
    a = o[:, 0:1]
    b = o[:, 1:2]
    m = jnp.maximum(a, b)
    lse = m + jnp.log(jnp.exp(a - m) + jnp.exp(b - m))
    o_ref[...] = o - lse


def _vmem_call(body, n_in, out_shape):
    return pl.pallas_call(
        body,
        out_shape=out_shape,
        in_specs=[pl.BlockSpec(memory_space=pltpu.VMEM)] * n_in,
        out_specs=pl.BlockSpec(memory_space=pltpu.VMEM),
    )


@jax.jit
def kernel(x, edge_index, W1, b1, W2, b2):
    src = edge_index[0].astype(jnp.int32)
    dst = edge_index[1].astype(jnp.int32)
    src_p = jnp.concatenate([src, jnp.zeros((E_PAD - E,), jnp.int32)])
    dst_p = jnp.concatenate([dst, jnp.full((E_PAD - E,), N, jnp.int32)])
    src_t = src_p.reshape(NW, CH, BATCH)
    dst_t = dst_p.reshape(NW, CH, BATCH)

    degp = _deg_kernel(dst_t)
    degp_t = degp.T[:N, :]

    g1 = _vmem_call(
        _tc_a_body, 3, jax.ShapeDtypeStruct((N, H), jnp.float32)
    )(x, W1, degp_t)

    s1 = _agg_kernel(g1, src_t, dst_t)

    q = _vmem_call(
        _tc_b_body, 4, jax.ShapeDtypeStruct((N, H), jnp.float32)
    )(degp_t, g1, s1, b1.reshape(1, H))

    r = _agg_kernel(q, src_t, dst_t)

    out = _vmem_call(
        _tc_c_body, 5, jax.ShapeDtypeStruct((N, OUT), jnp.float32)
    )(degp_t, q, r, W2, b2.reshape(1, OUT))
    return out


# speedup vs baseline: 21.7515x; 21.7515x over previous
import functools

import jax
import jax.numpy as jnp
from jax import lax
from jax.experimental import pallas as pl
from jax.experimental.pallas import tpu as pltpu
from jax.experimental.pallas import tpu_sc as plsc

N = 10000
E = 320000
D = 128
H = 32
OUT = 2

NC = 2
NS = 16
LANES = 16
NW = NC * NS

BATCH = 128
CH = 80
EDGES_PER_TILE = CH * BATCH
E_PAD = NW * EDGES_PER_TILE
N_PAD = 10240
RPT = N_PAD // NS

_mesh = plsc.VectorSubcoreMesh(
    core_axis_name="c", subcore_axis_name="s", num_cores=NC, num_subcores=NS
)
_sc_params = pltpu.CompilerParams(
    needs_layout_passes=False, use_tc_tiling_on_sc=False
)


@functools.partial(
    pl.kernel,
    out_type=jax.ShapeDtypeStruct((NC, N_PAD), jnp.float32),
    mesh=_mesh,
    compiler_params=_sc_params,
    scratch_types=[
        pltpu.VMEM((CH, BATCH), jnp.int32),
        pltpu.VMEM((N_PAD,), jnp.float32),
        pltpu.VMEM((NS, RPT), jnp.float32),
        pltpu.VMEM((RPT,), jnp.float32),
        pltpu.VMEM_SHARED((NS, N_PAD), jnp.float32),
    ],
)
def _deg_kernel(dst_hbm, out_hbm, dstbuf, degloc, stripe, outbuf, degsh):
    c = lax.axis_index("c")
    s = lax.axis_index("s")
    gid = c * NS + s

    pltpu.sync_copy(dst_hbm.at[gid], dstbuf)

    zeros16 = jnp.zeros((LANES,), jnp.float32)

    def _zero(i, _):
        degloc[pl.ds(i * LANES, LANES)] = zeros16
        return 0

    lax.fori_loop(0, N_PAD // LANES, _zero, 0)

    ones16 = jnp.ones((LANES,), jnp.float32)

    def _count(i, _):
        for j in range(BATCH // LANES):
            idx = dstbuf[i, pl.ds(j * LANES, LANES)]
            plsc.addupdate_scatter(degloc, [idx], ones16)
        return 0

    lax.fori_loop(0, CH, _count, 0)

    pltpu.sync_copy(degloc, degsh.at[s])
    plsc.subcore_barrier()
    pltpu.sync_copy(degsh.at[:, pl.ds(s * RPT, RPT)], stripe)

    def _reduce(j, _):
        acc = stripe[0, pl.ds(j * LANES, LANES)]
        for t in range(1, NS):
            acc = acc + stripe[t, pl.ds(j * LANES, LANES)]
        outbuf[pl.ds(j * LANES, LANES)] = acc
        return 0

    lax.fori_loop(0, RPT // LANES, _reduce, 0)
    pltpu.sync_copy(outbuf, out_hbm.at[c, pl.ds(s * RPT, RPT)])


@functools.partial(
    pl.kernel,
    out_type=jax.ShapeDtypeStruct((NC, N_PAD, H), jnp.float32),
    mesh=_mesh,
    compiler_params=_sc_params,
    scratch_types=[
        pltpu.VMEM((CH, BATCH), jnp.int32),
        pltpu.VMEM((CH, BATCH), jnp.int32),
        pltpu.VMEM((BATCH, H), jnp.float32),
        pltpu.VMEM((BATCH, H), jnp.float32),
        pltpu.VMEM((RPT, H), jnp.float32),
        pltpu.VMEM_SHARED((N_PAD, H), jnp.float32),
        pltpu.SemaphoreType.DMA,
        pltpu.SemaphoreType.DMA,
    ],
)
def _agg_kernel(g_hbm, src_hbm, dst_hbm, out_hbm,
                srcbuf, dstbuf, rows0, rows1, zbuf, acc, sem0, sem1):
    c = lax.axis_index("c")
    s = lax.axis_index("s")
    gid = c * NS + s

    pltpu.sync_copy(src_hbm.at[gid], srcbuf)
    pltpu.sync_copy(dst_hbm.at[gid], dstbuf)

    zeros16 = jnp.zeros((LANES,), jnp.float32)

    def _zero(i, _):
        zbuf[i, pl.ds(0, LANES)] = zeros16
        zbuf[i, pl.ds(LANES, LANES)] = zeros16
        return 0

    lax.fori_loop(0, RPT, _zero, 0)
    pltpu.sync_copy(zbuf, acc.at[pl.ds(s * RPT, RPT)])
    plsc.subcore_barrier()

    pltpu.async_copy(g_hbm.at[srcbuf.at[0]], rows0, sem0).wait()

    def _step(k, _):
        pltpu.async_copy(g_hbm.at[srcbuf.at[2 * k + 1]], rows1, sem1)
        pltpu.sync_copy(rows0, acc.at[dstbuf.at[2 * k]], add=True)
        pltpu.make_async_copy(g_hbm.at[srcbuf.at[2 * k + 1]], rows1, sem1).wait()
        pltpu.async_copy(g_hbm.at[srcbuf.at[2 * k + 2]], rows0, sem0)
        pltpu.sync_copy(rows1, acc.at[dstbuf.at[2 * k + 1]], add=True)
        pltpu.make_async_copy(g_hbm.at[srcbuf.at[2 * k + 2]], rows0, sem0).wait()
        return 0

    lax.fori_loop(0, CH // 2 - 1, _step, 0)
    pltpu.async_copy(g_hbm.at[srcbuf.at[CH - 1]], rows1, sem1)
    pltpu.sync_copy(rows0, acc.at[dstbuf.at[CH - 2]], add=True)
    pltpu.make_async_copy(g_hbm.at[srcbuf.at[CH - 1]], rows1, sem1).wait()
    pltpu.sync_copy(rows1, acc.at[dstbuf.at[CH - 1]], add=True)

    plsc.subcore_barrier()
    pltpu.sync_copy(acc.at[pl.ds(s * RPT, RPT)], out_hbm.at[c, pl.ds(s * RPT, RPT)])


def _dis_from(degp_ref):
    d = degp_ref[:, 0:1] + degp_ref[:, 1:2] + 1.0
    return lax.rsqrt(d)


def _tc_a_body(x_ref, w1_ref, degp_ref, g1_ref):
    dis = _dis_from(degp_ref)
    h = jnp.dot(x_ref[...], w1_ref[...], preferred_element_type=jnp.float32)
    g1_ref[...] = dis * h


def _tc_b_body(degp_ref, g1_ref, s1_ref, b1_ref, q_ref):
    dis = _dis_from(degp_ref)
    ssum = s1_ref[0, :N, :] + s1_ref[1, :N, :]
    out1 = dis * (ssum + g1_ref[...]) + b1_ref[...]
    q_ref[...] = dis * jnp.maximum(out1, 0.0)


def _tc_c_body(degp_ref, q_ref, r_ref, w2_ref, b2_ref, o_ref):
    dis = _dis_from(degp_ref)
    rsum = r_ref[0, :N, :] + r_ref[1, :N, :]
    t = dis * (rsum + q_ref[...])
    o = jnp.dot(t, w2_ref[...], preferred_element_type=jnp.float32) + b2_ref[...]
    a = o[:, 0:1]
    b = o[:, 1:2]
    m = jnp.maximum(a, b)
    lse = m + jnp.log(jnp.exp(a - m) + jnp.exp(b - m))
    o_ref[...] = o - lse


def _vmem_call(body, n_in, out_shape):
    return pl.pallas_call(
        body,
        out_shape=out_shape,
        in_specs=[pl.BlockSpec(memory_space=pltpu.VMEM)] * n_in,
        out_specs=pl.BlockSpec(memory_space=pltpu.VMEM),
    )


@jax.jit
def kernel(x, edge_index, W1, b1, W2, b2):
    src = edge_index[0].astype(jnp.int32)
    dst = edge_index[1].astype(jnp.int32)
    src_p = jnp.concatenate([src, jnp.zeros((E_PAD - E,), jnp.int32)])
    dst_p = jnp.concatenate([dst, jnp.full((E_PAD - E,), N, jnp.int32)])
    src_t = src_p.reshape(NW, CH, BATCH)
    dst_t = dst_p.reshape(NW, CH, BATCH)

    degp = _deg_kernel(dst_t)
    degp_t = degp.T[:N, :]

    g1 = _vmem_call(
        _tc_a_body, 3, jax.ShapeDtypeStruct((N, H), jnp.float32)
    )(x, W1, degp_t)

    s1 = _agg_kernel(g1, src_t, dst_t)

    q = _vmem_call(
        _tc_b_body, 4, jax.ShapeDtypeStruct((N, H), jnp.float32)
    )(degp_t, g1, s1, b1.reshape(1, H))

    r = _agg_kernel(q, src_t, dst_t)

    out = _vmem_call(
        _tc_c_body, 5, jax.ShapeDtypeStruct((N, OUT), jnp.float32)
    )(degp_t, q, r, W2, b2.reshape(1, OUT))
    return out


# speedup vs baseline: 24.7781x; 1.1391x over previous
import functools

import jax
import jax.numpy as jnp
from jax import lax
from jax.experimental import pallas as pl
from jax.experimental.pallas import tpu as pltpu
from jax.experimental.pallas import tpu_sc as plsc

N = 10000
E = 320000
D = 128
H = 32
OUT = 2

NC = 2
NS = 16
LANES = 16
NW = NC * NS

BATCH = 128
CH = 80
NBUF = 8
EDGES_PER_TILE = CH * BATCH
E_PAD = NW * EDGES_PER_TILE
N_PAD = 10240
RPT = N_PAD // NS

_mesh = plsc.VectorSubcoreMesh(
    core_axis_name="c", subcore_axis_name="s", num_cores=NC, num_subcores=NS
)
_sc_params = pltpu.CompilerParams(
    needs_layout_passes=False, use_tc_tiling_on_sc=False
)


@functools.partial(
    pl.kernel,
    out_type=jax.ShapeDtypeStruct((NC, N_PAD), jnp.float32),
    mesh=_mesh,
    compiler_params=_sc_params,
    scratch_types=[
        pltpu.VMEM((CH, BATCH), jnp.int32),
        pltpu.VMEM((N_PAD,), jnp.float32),
        pltpu.VMEM((NS, RPT), jnp.float32),
        pltpu.VMEM((RPT,), jnp.float32),
        pltpu.VMEM_SHARED((NS, N_PAD), jnp.float32),
    ],
)
def _deg_kernel(dst_hbm, out_hbm, dstbuf, degloc, stripe, outbuf, degsh):
    c = lax.axis_index("c")
    s = lax.axis_index("s")
    gid = c * NS + s

    pltpu.sync_copy(dst_hbm.at[gid], dstbuf)

    zeros16 = jnp.zeros((LANES,), jnp.float32)

    def _zero(i, _):
        degloc[pl.ds(i * LANES, LANES)] = zeros16
        return 0

    lax.fori_loop(0, N_PAD // LANES, _zero, 0)

    ones16 = jnp.ones((LANES,), jnp.float32)

    def _count(i, _):
        for j in range(BATCH // LANES):
            idx = dstbuf[i, pl.ds(j * LANES, LANES)]
            plsc.addupdate_scatter(degloc, [idx], ones16)
        return 0

    lax.fori_loop(0, CH, _count, 0)

    pltpu.sync_copy(degloc, degsh.at[s])
    plsc.subcore_barrier()
    pltpu.sync_copy(degsh.at[:, pl.ds(s * RPT, RPT)], stripe)

    def _reduce(j, _):
        acc = stripe[0, pl.ds(j * LANES, LANES)]
        for t in range(1, NS):
            acc = acc + stripe[t, pl.ds(j * LANES, LANES)]
        outbuf[pl.ds(j * LANES, LANES)] = acc
        return 0

    lax.fori_loop(0, RPT // LANES, _reduce, 0)
    pltpu.sync_copy(outbuf, out_hbm.at[c, pl.ds(s * RPT, RPT)])


@functools.partial(
    pl.kernel,
    out_type=jax.ShapeDtypeStruct((NC, N_PAD, H), jnp.float32),
    mesh=_mesh,
    compiler_params=_sc_params,
    scratch_types=[
        pltpu.VMEM((CH, BATCH), jnp.int32),
        pltpu.VMEM((CH, BATCH), jnp.int32),
        pltpu.VMEM((NBUF, BATCH, H), jnp.float32),
        pltpu.VMEM((RPT, H), jnp.float32),
        pltpu.VMEM_SHARED((N_PAD, H), jnp.float32),
        [pltpu.SemaphoreType.DMA] * NBUF,
        [pltpu.SemaphoreType.DMA] * NBUF,
    ],
)
def _agg_kernel(g_hbm, src_hbm, dst_hbm, out_hbm,
                srcbuf, dstbuf, rows, zbuf, acc, gsems, ssems):
    c = lax.axis_index("c")
    s = lax.axis_index("s")
    gid = c * NS + s

    pltpu.sync_copy(src_hbm.at[gid], srcbuf)
    pltpu.sync_copy(dst_hbm.at[gid], dstbuf)

    zeros16 = jnp.zeros((LANES,), jnp.float32)

    def _zero(i, _):
        zbuf[i, pl.ds(0, LANES)] = zeros16
        zbuf[i, pl.ds(LANES, LANES)] = zeros16
        return 0

    lax.fori_loop(0, RPT, _zero, 0)
    pltpu.sync_copy(zbuf, acc.at[pl.ds(s * RPT, RPT)])
    plsc.subcore_barrier()

    for b in range(NBUF):
        pltpu.async_copy(g_hbm.at[srcbuf.at[b]], rows.at[b], gsems[b])

    def _round(it, _):
        k0 = it * NBUF
        for b in range(NBUF):
            pltpu.make_async_copy(g_hbm.at[srcbuf.at[k0 + b]], rows.at[b],
                                  gsems[b]).wait()
            pltpu.async_copy(rows.at[b], acc.at[dstbuf.at[k0 + b]], ssems[b],
                             add=True)
        for b in range(NBUF):
            pltpu.make_async_copy(rows.at[b], acc.at[dstbuf.at[k0 + b]],
                                  ssems[b]).wait()
            pltpu.async_copy(g_hbm.at[srcbuf.at[k0 + NBUF + b]], rows.at[b],
                             gsems[b])
        return 0

    lax.fori_loop(0, CH // NBUF - 1, _round, 0)
    k0 = CH - NBUF
    for b in range(NBUF):
        pltpu.make_async_copy(g_hbm.at[srcbuf.at[k0 + b]], rows.at[b],
                              gsems[b]).wait()
        pltpu.async_copy(rows.at[b], acc.at[dstbuf.at[k0 + b]], ssems[b],
                         add=True)
    for b in range(NBUF):
        pltpu.make_async_copy(rows.at[b], acc.at[dstbuf.at[k0 + b]],
                              ssems[b]).wait()

    plsc.subcore_barrier()
    pltpu.sync_copy(acc.at[pl.ds(s * RPT, RPT)], out_hbm.at[c, pl.ds(s * RPT, RPT)])


def _dis_from(degp_ref):
    d = degp_ref[:, 0:1] + degp_ref[:, 1:2] + 1.0
    return lax.rsqrt(d)


def _tc_a_body(x_ref, w1_ref, degp_ref, g1_ref):
    dis = _dis_from(degp_ref)
    h = jnp.dot(x_ref[...], w1_ref[...], preferred_element_type=jnp.float32)
    g1_ref[...] = dis * h


def _tc_b_body(degp_ref, g1_ref, s1_ref, b1_ref, q_ref):
    dis = _dis_from(degp_ref)
    ssum = s1_ref[0, :N, :] + s1_ref[1, :N, :]
    out1 = dis * (ssum + g1_ref[...]) + b1_ref[...]
    q_ref[...] = dis * jnp.maximum(out1, 0.0)


def _tc_c_body(degp_ref, q_ref, r_ref, w2_ref, b2_ref, o_ref):
    dis = _dis_from(degp_ref)
    rsum = r_ref[0, :N, :] + r_ref[1, :N, :]
    t = dis * (rsum + q_ref[...])
    o = jnp.dot(t, w2_ref[...], preferred_element_type=jnp.float32) + b2_ref[...]
    a = o[:, 0:1]
    b = o[:, 1:2]
    m = jnp.maximum(a, b)
    lse = m + jnp.log(jnp.exp(a - m) + jnp.exp(b - m))
    o_ref[...] = o - lse


def _vmem_call(body, n_in, out_shape):
    return pl.pallas_call(
        body,
        out_shape=out_shape,
        in_specs=[pl.BlockSpec(memory_space=pltpu.VMEM)] * n_in,
        out_specs=pl.BlockSpec(memory_space=pltpu.VMEM),
    )


@jax.jit
def kernel(x, edge_index, W1, b1, W2, b2):
    src = edge_index[0].astype(jnp.int32)
    dst = edge_index[1].astype(jnp.int32)
    src_p = jnp.concatenate([src, jnp.zeros((E_PAD - E,), jnp.int32)])
    dst_p = jnp.concatenate([dst, jnp.full((E_PAD - E,), N, jnp.int32)])
    src_t = src_p.reshape(NW, CH, BATCH)
    dst_t = dst_p.reshape(NW, CH, BATCH)

    degp = _deg_kernel(dst_t)
    degp_t = degp.T[:N, :]

    g1 = _vmem_call(
        _tc_a_body, 3, jax.ShapeDtypeStruct((N, H), jnp.float32)
    )(x, W1, degp_t)

    s1 = _agg_kernel(g1, src_t, dst_t)

    q = _vmem_call(
        _tc_b_body, 4, jax.ShapeDtypeStruct((N, H), jnp.float32)
    )(degp_t, g1, s1, b1.reshape(1, H))

    r = _agg_kernel(q, src_t, dst_t)

    out = _vmem_call(
        _tc_c_body, 5, jax.ShapeDtypeStruct((N, OUT), jnp.float32)
    )(degp_t, q, r, W2, b2.reshape(1, OUT))
    return out


# speedup vs baseline: 25.1937x; 1.0168x over previous
import functools

import jax
import jax.numpy as jnp
from jax import lax
from jax.experimental import pallas as pl
from jax.experimental.pallas import tpu as pltpu
from jax.experimental.pallas import tpu_sc as plsc

N = 10000
E = 320000
D = 128
H = 32
OUT = 2

NC = 2
NS = 16
LANES = 16
NW = NC * NS

BATCH = 128
CH = 80
NBUF = 8
EDGES_PER_TILE = CH * BATCH
E_PAD = NW * EDGES_PER_TILE
N_PAD = 10240
RPT = N_PAD // NS

_mesh = plsc.VectorSubcoreMesh(
    core_axis_name="c", subcore_axis_name="s", num_cores=NC, num_subcores=NS
)
_sc_params = pltpu.CompilerParams(
    needs_layout_passes=False, use_tc_tiling_on_sc=False
)


@functools.partial(
    pl.kernel,
    out_type=jax.ShapeDtypeStruct((NC, N_PAD), jnp.float32),
    mesh=_mesh,
    compiler_params=_sc_params,
    scratch_types=[
        pltpu.VMEM((CH, BATCH), jnp.int32),
        pltpu.VMEM((N_PAD,), jnp.float32),
        pltpu.VMEM((NS, RPT), jnp.float32),
        pltpu.VMEM((RPT,), jnp.float32),
        pltpu.VMEM_SHARED((NS, N_PAD), jnp.float32),
    ],
)
def _deg_kernel(dst_hbm, out_hbm, dstbuf, degloc, stripe, outbuf, degsh):
    c = lax.axis_index("c")
    s = lax.axis_index("s")
    gid = c * NS + s

    pltpu.sync_copy(dst_hbm.at[gid], dstbuf)

    zeros16 = jnp.zeros((LANES,), jnp.float32)

    def _zero(i, _):
        degloc[pl.ds(i * LANES, LANES)] = zeros16
        return 0

    lax.fori_loop(0, N_PAD // LANES, _zero, 0)

    ones16 = jnp.ones((LANES,), jnp.float32)

    def _count(i, _):
        for j in range(BATCH // LANES):
            idx = dstbuf[i, pl.ds(j * LANES, LANES)]
            plsc.addupdate_scatter(degloc, [idx], ones16)
        return 0

    lax.fori_loop(0, CH, _count, 0)

    pltpu.sync_copy(degloc, degsh.at[s])
    plsc.subcore_barrier()
    pltpu.sync_copy(degsh.at[:, pl.ds(s * RPT, RPT)], stripe)

    def _reduce(j, _):
        acc = stripe[0, pl.ds(j * LANES, LANES)]
        for t in range(1, NS):
            acc = acc + stripe[t, pl.ds(j * LANES, LANES)]
        outbuf[pl.ds(j * LANES, LANES)] = acc
        return 0

    lax.fori_loop(0, RPT // LANES, _reduce, 0)
    pltpu.sync_copy(outbuf, out_hbm.at[c, pl.ds(s * RPT, RPT)])


@functools.partial(
    pl.kernel,
    out_type=jax.ShapeDtypeStruct((NC, N_PAD, H), jnp.float32),
    mesh=_mesh,
    compiler_params=_sc_params,
    scratch_types=[
        pltpu.VMEM((CH, BATCH), jnp.int32),
        pltpu.VMEM((CH, BATCH), jnp.int32),
        pltpu.VMEM((NBUF, BATCH, H), jnp.float32),
        pltpu.VMEM((RPT, H), jnp.float32),
        pltpu.VMEM_SHARED((N_PAD, H), jnp.float32),
        [pltpu.SemaphoreType.DMA] * NBUF,
        [pltpu.SemaphoreType.DMA] * NBUF,
    ],
)
def _agg_kernel(g_hbm, src_hbm, dst_hbm, out_hbm,
                srcbuf, dstbuf, rows, zbuf, acc, gsems, ssems):
    c = lax.axis_index("c")
    s = lax.axis_index("s")
    gid = c * NS + s

    pltpu.sync_copy(src_hbm.at[gid], srcbuf)
    pltpu.sync_copy(dst_hbm.at[gid], dstbuf)

    zeros16 = jnp.zeros((LANES,), jnp.float32)

    def _zero(i, _):
        zbuf[i, pl.ds(0, LANES)] = zeros16
        zbuf[i, pl.ds(LANES, LANES)] = zeros16
        return 0

    lax.fori_loop(0, RPT, _zero, 0)
    pltpu.sync_copy(zbuf, acc.at[pl.ds(s * RPT, RPT)])
    plsc.subcore_barrier()

    for b in range(NBUF):
        pltpu.async_copy(g_hbm.at[srcbuf.at[b]], rows.at[b], gsems[b])

    def _round(it, _):
        k0 = it * NBUF
        for b in range(NBUF):
            pltpu.make_async_copy(g_hbm.at[srcbuf.at[k0 + b]], rows.at[b],
                                  gsems[b]).wait()
            pltpu.async_copy(rows.at[b], acc.at[dstbuf.at[k0 + b]], ssems[b],
                             add=True)
        for b in range(NBUF):
            pltpu.make_async_copy(rows.at[b], acc.at[dstbuf.at[k0 + b]],
                                  ssems[b]).wait()
            pltpu.async_copy(g_hbm.at[srcbuf.at[k0 + NBUF + b]], rows.at[b],
                             gsems[b])
        return 0

    lax.fori_loop(0, CH // NBUF - 1, _round, 0)
    k0 = CH - NBUF
    for b in range(NBUF):
        pltpu.make_async_copy(g_hbm.at[srcbuf.at[k0 + b]], rows.at[b],
                              gsems[b]).wait()
        pltpu.async_copy(rows.at[b], acc.at[dstbuf.at[k0 + b]], ssems[b],
                         add=True)
    for b in range(NBUF):
        pltpu.make_async_copy(rows.at[b], acc.at[dstbuf.at[k0 + b]],
                              ssems[b]).wait()

    plsc.subcore_barrier()
    pltpu.sync_copy(acc.at[pl.ds(s * RPT, RPT)], out_hbm.at[c, pl.ds(s * RPT, RPT)])


def _dis_from(degp_ref):
    d = degp_ref[:, 0:1] + degp_ref[:, 1:2] + 1.0
    return lax.rsqrt(d)


def _tc_a_body(x_ref, w1_ref, degp_ref, g1_ref):
    dis = _dis_from(degp_ref)
    h = jnp.dot(x_ref[...], w1_ref[...], preferred_element_type=jnp.float32)
    g1_ref[...] = dis * h


def _tc_b_body(degp_ref, g1_ref, s1_ref, b1_ref, q_ref):
    dis = _dis_from(degp_ref)
    ssum = s1_ref[0, :N, :] + s1_ref[1, :N, :]
    out1 = dis * (ssum + g1_ref[...]) + b1_ref[...]
    q_ref[...] = dis * jnp.maximum(out1, 0.0)


def _tc_c_body(degp_ref, q_ref, r_ref, w2_ref, b2_ref, o_ref):
    dis = _dis_from(degp_ref)
    rsum = r_ref[0, :N, :] + r_ref[1, :N, :]
    t = dis * (rsum + q_ref[...])
    o = jnp.dot(t, w2_ref[...], preferred_element_type=jnp.float32) + b2_ref[...]
    a = o[:, 0:1]
    b = o[:, 1:2]
    m = jnp.maximum(a, b)
    lse = m + jnp.log(jnp.exp(a - m) + jnp.exp(b - m))
    o_ref[...] = o - lse


def _vmem_call(body, n_in, out_shape):
    return pl.pallas_call(
        body,
        out_shape=out_shape,
        in_specs=[pl.BlockSpec(memory_space=pltpu.VMEM)] * n_in,
        out_specs=pl.BlockSpec(memory_space=pltpu.VMEM),
    )


@jax.jit
def kernel(x, edge_index, W1, b1, W2, b2):
    src = edge_index[0].astype(jnp.int32)
    dst = edge_index[1].astype(jnp.int32)
    src_p = jnp.concatenate([src, jnp.zeros((E_PAD - E,), jnp.int32)])
    pad_dst = N + jnp.arange(E_PAD - E, dtype=jnp.int32) % (N_PAD - N)
    dst_p = jnp.concatenate([dst, pad_dst])
    src_t = src_p.reshape(NW, CH, BATCH)
    dst_t = dst_p.reshape(NW, CH, BATCH)

    degp = _deg_kernel(dst_t)
    degp_t = degp.T[:N, :]

    g1 = _vmem_call(
        _tc_a_body, 3, jax.ShapeDtypeStruct((N, H), jnp.float32)
    )(x, W1, degp_t)

    s1 = _agg_kernel(g1, src_t, dst_t)

    q = _vmem_call(
        _tc_b_body, 4, jax.ShapeDtypeStruct((N, H), jnp.float32)
    )(degp_t, g1, s1, b1.reshape(1, H))

    r = _agg_kernel(q, src_t, dst_t)

    out = _vmem_call(
        _tc_c_body, 5, jax.ShapeDtypeStruct((N, OUT), jnp.float32)
    )(degp_t, q, r, W2, b2.reshape(1, OUT))
    return out


# speedup vs baseline: 52.3457x; 2.0777x over previous
import functools

import jax
import jax.numpy as jnp
from jax import lax
from jax.experimental import pallas as pl
from jax.experimental.pallas import tpu as pltpu
from jax.experimental.pallas import tpu_sc as plsc

N = 10000
E = 320000
D = 128
H = 32
OUT = 2

NC = 2
NS = 16
LANES = 16
NW = NC * NS

BATCH = 128
CH = 80
NBUF = 8
EDGES_PER_TILE = CH * BATCH
E_PAD = NW * EDGES_PER_TILE
N_PAD = 10240
RPT = N_PAD // NS

_mesh = plsc.VectorSubcoreMesh(
    core_axis_name="c", subcore_axis_name="s", num_cores=NC, num_subcores=NS
)
_sc_params = pltpu.CompilerParams(
    needs_layout_passes=False, use_tc_tiling_on_sc=False
)


@functools.partial(
    pl.kernel,
    out_type=jax.ShapeDtypeStruct((NC, N_PAD), jnp.float32),
    mesh=_mesh,
    compiler_params=_sc_params,
    scratch_types=[
        pltpu.VMEM((CH, BATCH), jnp.int32),
        pltpu.VMEM((N_PAD,), jnp.float32),
        pltpu.VMEM((NS, RPT), jnp.float32),
        pltpu.VMEM((RPT,), jnp.float32),
        pltpu.VMEM_SHARED((NS, N_PAD), jnp.float32),
    ],
)
def _deg_kernel(dst_hbm, out_hbm, dstbuf, degloc, stripe, outbuf, degsh):
    c = lax.axis_index("c")
    s = lax.axis_index("s")
    gid = c * NS + s

    pltpu.sync_copy(dst_hbm.at[gid], dstbuf)

    zeros16 = jnp.zeros((LANES,), jnp.float32)

    def _zero(i, _):
        degloc[pl.ds(i * LANES, LANES)] = zeros16
        return 0

    lax.fori_loop(0, N_PAD // LANES, _zero, 0)

    ones16 = jnp.ones((LANES,), jnp.float32)

    def _count(i, _):
        for j in range(BATCH // LANES):
            idx = dstbuf[i, pl.ds(j * LANES, LANES)]
            plsc.addupdate_scatter(degloc, [idx], ones16)
        return 0

    lax.fori_loop(0, CH, _count, 0)

    pltpu.sync_copy(degloc, degsh.at[s])
    plsc.subcore_barrier()
    pltpu.sync_copy(degsh.at[:, pl.ds(s * RPT, RPT)], stripe)

    def _reduce(j, _):
        acc = stripe[0, pl.ds(j * LANES, LANES)]
        for t in range(1, NS):
            acc = acc + stripe[t, pl.ds(j * LANES, LANES)]
        outbuf[pl.ds(j * LANES, LANES)] = acc
        return 0

    lax.fori_loop(0, RPT // LANES, _reduce, 0)
    pltpu.sync_copy(outbuf, out_hbm.at[c, pl.ds(s * RPT, RPT)])


@functools.partial(
    pl.kernel,
    out_type=jax.ShapeDtypeStruct((NC, N_PAD, H), jnp.float32),
    mesh=_mesh,
    compiler_params=_sc_params,
    scratch_types=[
        pltpu.VMEM((CH, BATCH), jnp.int32),
        pltpu.VMEM((CH, BATCH), jnp.int32),
        pltpu.VMEM((NBUF, BATCH, H), jnp.float32),
        pltpu.VMEM((RPT, H), jnp.float32),
        pltpu.VMEM_SHARED((N_PAD, H), jnp.float32),
        [pltpu.SemaphoreType.DMA] * NBUF,
        [pltpu.SemaphoreType.DMA] * NBUF,
    ],
)
def _agg_kernel(g_hbm, src_hbm, dst_hbm, out_hbm,
                srcbuf, dstbuf, rows, zbuf, acc, gsems, ssems):
    c = lax.axis_index("c")
    s = lax.axis_index("s")
    gid = c * NS + s

    pltpu.sync_copy(src_hbm.at[gid], srcbuf)
    pltpu.sync_copy(dst_hbm.at[gid], dstbuf)

    zeros16 = jnp.zeros((LANES,), jnp.float32)

    def _zero(i, _):
        zbuf[i, pl.ds(0, LANES)] = zeros16
        zbuf[i, pl.ds(LANES, LANES)] = zeros16
        return 0

    lax.fori_loop(0, RPT, _zero, 0)
    pltpu.sync_copy(zbuf, acc.at[pl.ds(s * RPT, RPT)])
    plsc.subcore_barrier()

    for b in range(NBUF):
        pltpu.async_copy(g_hbm.at[srcbuf.at[b]], rows.at[b], gsems[b])

    def _round(it, _):
        k0 = it * NBUF
        for b in range(NBUF):
            pltpu.make_async_copy(g_hbm.at[srcbuf.at[k0 + b]], rows.at[b],
                                  gsems[b]).wait()
            pltpu.async_copy(rows.at[b], acc.at[dstbuf.at[k0 + b]], ssems[b],
                             add=True)
        for b in range(NBUF):
            pltpu.make_async_copy(rows.at[b], acc.at[dstbuf.at[k0 + b]],
                                  ssems[b]).wait()
            pltpu.async_copy(g_hbm.at[srcbuf.at[k0 + NBUF + b]], rows.at[b],
                             gsems[b])
        return 0

    lax.fori_loop(0, CH // NBUF - 1, _round, 0)
    k0 = CH - NBUF
    for b in range(NBUF):
        pltpu.make_async_copy(g_hbm.at[srcbuf.at[k0 + b]], rows.at[b],
                              gsems[b]).wait()
        pltpu.async_copy(rows.at[b], acc.at[dstbuf.at[k0 + b]], ssems[b],
                         add=True)
    for b in range(NBUF):
        pltpu.make_async_copy(rows.at[b], acc.at[dstbuf.at[k0 + b]],
                              ssems[b]).wait()

    plsc.subcore_barrier()
    pltpu.sync_copy(acc.at[pl.ds(s * RPT, RPT)], out_hbm.at[c, pl.ds(s * RPT, RPT)])


def _dis_from(degp_ref):
    d = degp_ref[:, 0:1] + degp_ref[:, 1:2] + 1.0
    return lax.rsqrt(d)


def _tc_a_body(x_ref, w1_ref, degp_ref, g1_ref):
    dis = _dis_from(degp_ref)
    h = jnp.dot(x_ref[...], w1_ref[...], preferred_element_type=jnp.float32)
    g1_ref[...] = dis * h


def _tc_b_body(degp_ref, g1_ref, s1_ref, b1_ref, q_ref):
    dis = _dis_from(degp_ref)
    ssum = s1_ref[0, :N, :] + s1_ref[1, :N, :]
    out1 = dis * (ssum + g1_ref[...]) + b1_ref[...]
    q_ref[...] = dis * jnp.maximum(out1, 0.0)


def _tc_c_body(degp_ref, q_ref, r_ref, w2_ref, b2_ref, o_ref):
    dis = _dis_from(degp_ref)
    rsum = r_ref[0, :N, :] + r_ref[1, :N, :]
    t = dis * (rsum + q_ref[...])
    o = jnp.dot(t, w2_ref[...], preferred_element_type=jnp.float32) + b2_ref[...]
    a = o[:, 0:1]
    b = o[:, 1:2]
    m = jnp.maximum(a, b)
    lse = m + jnp.log(jnp.exp(a - m) + jnp.exp(b - m))
    o_ref[...] = o - lse


def _vmem_call(body, n_in, out_shape):
    return pl.pallas_call(
        body,
        out_shape=out_shape,
        in_specs=[pl.BlockSpec(memory_space=pltpu.VMEM)] * n_in,
        out_specs=pl.BlockSpec(memory_space=pltpu.VMEM),
    )


@jax.jit
def kernel(x, edge_index, W1, b1, W2, b2):
    src = edge_index[0].astype(jnp.int32)
    dst = edge_index[1].astype(jnp.int32)
    pad_iota = jnp.arange(E_PAD - E, dtype=jnp.int32)
    src_p = jnp.concatenate([src, pad_iota % N])
    dst_p = jnp.concatenate([dst, N + pad_iota % (N_PAD - N)])
    src_t = src_p.reshape(NW, CH, BATCH)
    dst_t = dst_p.reshape(NW, CH, BATCH)

    degp = _deg_kernel(dst_t)
    degp_t = degp.T[:N, :]

    g1 = _vmem_call(
        _tc_a_body, 3, jax.ShapeDtypeStruct((N, H), jnp.float32)
    )(x, W1, degp_t)

    s1 = _agg_kernel(g1, src_t, dst_t)

    q = _vmem_call(
        _tc_b_body, 4, jax.ShapeDtypeStruct((N, H), jnp.float32)
    )(degp_t, g1, s1, b1.reshape(1, H))

    r = _agg_kernel(q, src_t, dst_t)

    out = _vmem_call(
        _tc_c_body, 5, jax.ShapeDtypeStruct((N, OUT), jnp.float32)
    )(degp_t, q, r, W2, b2.reshape(1, OUT))
    return out
